# Initial kernel scaffold; baseline (speedup 1.0000x reference)
import functools

import jax
import jax.numpy as jnp
from jax import lax
from jax.experimental import pallas as pl
from jax.experimental.pallas import tpu as pltpu
from jax.experimental.pallas import tpu_sc as plsc

N_REAL = 10000
NP = 10240
K = 8
F_IN = 128
H = 64
O_DIM = 32
BR = 256
NBLK = NP // BR
NW = 32
RPW = NP // NW
EPW = RPW * K
GPW = EPW // 16
TPS = 16
RPT = NP // TPS

_NEG = jnp.float32(-3e38)


def _gru_body(x_ref, *refs):
    (wir_s, whr_s, br_s, bhr_s, wiz_s, whz_s, bz_s, bhz_s, win_s, whn_s,
     bn_s, bhn_s,
     wir_o, whr_o, br_o, bhr_o, wiz_o, whz_o, bz_o, bhz_o, win_o, whn_o,
     bn_o, bhn_o,
     w1, hs_ref, xw1_ref) = refs

    def one_gru(wir, whr, bir, bhr, wiz, whz, biz, bhz, win, whn, bin_, bhn):
        h = jnp.zeros((BR, H), jnp.float32)
        for l in range(4):
            xl = x_ref[l]
            dn = (((0,), (1,)), ((), ()))
            gir = lax.dot_general(xl, wir[...], dn) + bir[...]
            giz = lax.dot_general(xl, wiz[...], dn) + biz[...]
            gin = lax.dot_general(xl, win[...], dn) + bin_[...]
            dnh = (((1,), (1,)), ((), ()))
            ghr = lax.dot_general(h, whr[...], dnh) + bhr[...]
            ghz = lax.dot_general(h, whz[...], dnh) + bhz[...]
            ghn = lax.dot_general(h, whn[...], dnh) + bhn[...]
            r = jax.nn.sigmoid(gir + ghr)
            z = jax.nn.sigmoid(giz + ghz)
            n = jnp.tanh(gin + r * ghn)
            h = (1.0 - z) * n + z * h
        return h

    h_sim = one_gru(wir_s, whr_s, br_s, bhr_s, wiz_s, whz_s, bz_s, bhz_s,
                    win_s, whn_s, bn_s, bhn_s)
    h_out = one_gru(wir_o, whr_o, br_o, bhr_o, wiz_o, whz_o, bz_o, bhz_o,
                    win_o, whn_o, bn_o, bhn_o)
    hs_ref[...] = h_sim
    xw1_ref[...] = lax.dot_general(h_out, w1[...], (((1,), (1,)), ((), ())))


def _run_gru(xp, wsets, w1):
    full = lambda a: pl.BlockSpec(a.shape, lambda i: (0,) * a.ndim)
    in_specs = [pl.BlockSpec((4, F_IN, BR), lambda i: (0, 0, i))]
    in_specs += [full(a) for a in wsets] + [full(w1)]
    out_specs = [pl.BlockSpec((BR, H), lambda i: (i, 0))] * 2
    return pl.pallas_call(
        _gru_body,
        grid=(NBLK,),
        in_specs=in_specs,
        out_specs=out_specs,
        out_shape=[jax.ShapeDtypeStruct((NP, H), jnp.float32)] * 2,
    )(xp, *wsets, w1)


def _simtopk_body(hrow_ref, hall_ref, vals_ref, idx_ref, sbuf_ref):
    i = pl.program_id(0)
    hrow = hrow_ref[...]
    bc = 2048
    for c in range(NP // bc):
        hc = hall_ref[pl.ds(c * bc, bc), :]
        s = lax.dot_general(hrow, hc, (((1,), (1,)), ((), ())))
        col = lax.broadcasted_iota(jnp.int32, (BR, bc), 1) + c * bc
        row = lax.broadcasted_iota(jnp.int32, (BR, bc), 0) + i * BR
        s = jnp.where(col == row, jnp.float32(-1e9), s)
        s = jnp.where(col >= N_REAL, _NEG, s)
        sbuf_ref[:, pl.ds(c * bc, bc)] = s

    coln = lax.broadcasted_iota(jnp.int32, (BR, NP), 1)
    lane8 = lax.broadcasted_iota(jnp.int32, (BR, K), 1)
    vals8 = jnp.zeros((BR, K), jnp.float32)
    idx8 = jnp.zeros((BR, K), jnp.int32)
    for t in range(K):
        sb = sbuf_ref[...]
        m = jnp.max(sb, axis=1, keepdims=True)
        ism = sb == m
        pos = jnp.min(jnp.where(ism, coln, jnp.int32(2 ** 30)), axis=1,
                      keepdims=True)
        vals8 = jnp.where(lane8 == t, m, vals8)
        idx8 = jnp.where(lane8 == t, pos, idx8)
        if t < K - 1:
            sbuf_ref[...] = jnp.where(coln == pos, _NEG, sb)
    rowv = lax.broadcasted_iota(jnp.int32, (BR, K), 0) + i * BR
    valid = rowv < N_REAL
    vals_ref[...] = jnp.where(valid, vals8, 0.0)
    idx_ref[...] = jnp.where(valid, idx8, 0)


def _run_simtopk(h_sim):
    return pl.pallas_call(
        _simtopk_body,
        grid=(NBLK,),
        in_specs=[pl.BlockSpec((BR, H), lambda i: (i, 0)),
                  pl.BlockSpec((NP, H), lambda i: (0, 0))],
        out_specs=[pl.BlockSpec((BR, K), lambda i: (i, 0)),
                   pl.BlockSpec((BR, K), lambda i: (i, 0))],
        out_shape=[jax.ShapeDtypeStruct((NP, K), jnp.float32),
                   jax.ShapeDtypeStruct((NP, K), jnp.int32)],
        scratch_shapes=[pltpu.VMEM((BR, NP), jnp.float32)],
    )(h_sim, h_sim)


def _deg_body(idx_hbm, vals_hbm, out_hbm, idx_v, vals_v, deg_v):
    cid = lax.axis_index("c")
    sid = lax.axis_index("s")
    wid = cid * TPS + sid
    base = wid * EPW
    pltpu.sync_copy(idx_hbm.at[pl.ds(base, EPW)], idx_v)
    pltpu.sync_copy(vals_hbm.at[pl.ds(base, EPW)], vals_v)
    zero = jnp.zeros((16,), jnp.float32)

    def zb(j, carry):
        deg_v[pl.ds(j * 16, 16)] = zero
        return carry
    lax.fori_loop(0, NP // 16, zb, 0)

    lane = lax.iota(jnp.int32, 16)
    mlo = lane < 8
    mhi = lane >= 8

    def body(g, carry):
        d16 = idx_v[pl.ds(g * 16, 16)]
        w16 = vals_v[pl.ds(g * 16, 16)]
        plsc.addupdate_scatter(deg_v, [d16], w16, mask=mlo)
        plsc.addupdate_scatter(deg_v, [d16], w16, mask=mhi)
        return carry
    lax.fori_loop(0, GPW, body, 0)
    pltpu.sync_copy(deg_v, out_hbm.at[wid])


def _run_deg(idx_flat, vals_flat):
    mesh = plsc.VectorSubcoreMesh(core_axis_name="c", subcore_axis_name="s")
    kfn = pl.kernel(
        _deg_body,
        out_type=jax.ShapeDtypeStruct((NW, NP), jnp.float32),
        mesh=mesh,
        scratch_types=[pltpu.VMEM((EPW,), jnp.int32),
                       pltpu.VMEM((EPW,), jnp.float32),
                       pltpu.VMEM((NP,), jnp.float32)],
    )
    return kfn(idx_flat, vals_flat)


def _dinv_body(degp_ref, dinv_ref):
    j = pl.program_id(0)
    dsum = jnp.sum(degp_ref[...], axis=0, keepdims=True)
    col = lax.broadcasted_iota(jnp.int32, (1, 1024), 1) + j * 1024
    deg = dsum + jnp.where(col < N_REAL, 1.0, 0.0)
    dinv_ref[...] = jnp.where(deg > 0, lax.rsqrt(deg), 0.0)


def _run_dinv(deg_p):
    return pl.pallas_call(
        _dinv_body,
        grid=(NP // 1024,),
        in_specs=[pl.BlockSpec((NW, 1024), lambda j: (0, j))],
        out_specs=pl.BlockSpec((1, 1024), lambda j: (0, j)),
        out_shape=jax.ShapeDtypeStruct((1, NP), jnp.float32),
    )(deg_p)


def _msg_body(xw_hbm, idx_hbm, vals_hbm, dinv_hbm, out_hbm,
              xw_v, idx_v, vals_v, dinv_v, msg_v, scale_v, z_v, acc):
    cid = lax.axis_index("c")
    sid = lax.axis_index("s")
    wid = cid * TPS + sid
    rbase = wid * RPW
    ebase = wid * EPW
    pltpu.sync_copy(xw_hbm.at[pl.ds(rbase, RPW)], xw_v)
    pltpu.sync_copy(idx_hbm.at[pl.ds(ebase, EPW)], idx_v)
    pltpu.sync_copy(vals_hbm.at[pl.ds(ebase, EPW)], vals_v)
    pltpu.sync_copy(dinv_hbm, dinv_v)

    zero = jnp.zeros((16,), jnp.float32)
    for r in range(16):
        for j in range(H // 16):
            z_v[r, pl.ds(j * 16, 16)] = zero
    abase = sid * RPT

    def zb(j, carry):
        pltpu.sync_copy(z_v, acc.at[pl.ds(abase + j * 16, 16)])
        return carry
    lax.fori_loop(0, RPT // 16, zb, 0)
    plsc.subcore_barrier()

    lane = lax.iota(jnp.int32, 16)

    def body(g, carry):
        e0 = g * 16
        d16 = idx_v[pl.ds(e0, 16)]
        w16 = vals_v[pl.ds(e0, 16)]
        srcg = rbase + lax.shift_right_logical(e0 + lane, 3)
        dsrc = plsc.load_gather(dinv_v, [srcg])
        ddst = plsc.load_gather(dinv_v, [d16])
        scale_v[...] = dsrc * w16 * ddst
        for half in range(2):
            r = g * 2 + half
            xr = [xw_v[r, pl.ds(j * 16, 16)] for j in range(H // 16)]
            for e8 in range(8):
                e = half * 8 + e8
                sc = plsc.load_gather(scale_v,
                                      [jnp.full((16,), e, jnp.int32)])
                for j in range(H // 16):
                    msg_v[e, pl.ds(j * 16, 16)] = xr[j] * sc
        pltpu.sync_copy(msg_v, acc.at[d16], add=True)
        return carry
    lax.fori_loop(0, GPW, body, 0)
    plsc.subcore_barrier()
    pltpu.sync_copy(acc.at[pl.ds(abase, RPT)],
                    out_hbm.at[cid, pl.ds(abase, RPT)])


def _run_msg(xw, idx_flat, vals_flat, dinv):
    mesh = plsc.VectorSubcoreMesh(core_axis_name="c", subcore_axis_name="s")
    kfn = pl.kernel(
        _msg_body,
        out_type=jax.ShapeDtypeStruct((2, NP, H), jnp.float32),
        mesh=mesh,
        scratch_types=[pltpu.VMEM((RPW, H), jnp.float32),
                       pltpu.VMEM((EPW,), jnp.int32),
                       pltpu.VMEM((EPW,), jnp.float32),
                       pltpu.VMEM((NP,), jnp.float32),
                       pltpu.VMEM((16, H), jnp.float32),
                       pltpu.VMEM((16,), jnp.float32),
                       pltpu.VMEM((16, H), jnp.float32),
                       pltpu.VMEM_SHARED((NP, H), jnp.float32)],
    )
    return kfn(xw, idx_flat, vals_flat, dinv)


def _combine_body(p_ref, xw_ref, dinv_ref, b_ref, w_ref, out_ref):
    p = p_ref[0] + p_ref[1]
    dv = dinv_ref[...]
    o = p + dv * dv * xw_ref[...] + b_ref[...]
    o = jnp.maximum(o, 0.0)
    out_ref[...] = lax.dot_general(o, w_ref[...], (((1,), (1,)), ((), ())))


def _run_combine(partials, xw, dinv_col, b, w):
    return pl.pallas_call(
        _combine_body,
        grid=(NBLK,),
        in_specs=[pl.BlockSpec((2, BR, H), lambda i: (0, i, 0)),
                  pl.BlockSpec((BR, H), lambda i: (i, 0)),
                  pl.BlockSpec((BR, 1), lambda i: (i, 0)),
                  pl.BlockSpec((1, H), lambda i: (0, 0)),
                  pl.BlockSpec(w.shape, lambda i: (0, 0))],
        out_specs=pl.BlockSpec((BR, w.shape[0]), lambda i: (i, 0)),
        out_shape=jax.ShapeDtypeStruct((NP, w.shape[0]), jnp.float32),
    )(partials, xw, dinv_col, b, w)


def _final_relu_body(p_ref, xw_ref, dinv_ref, b_ref, out_ref, st_ref):
    i = pl.program_id(0)
    p = p_ref[0] + p_ref[1]
    dv = dinv_ref[...]
    o = p + dv * dv * xw_ref[...] + b_ref[...]
    o = jnp.maximum(o, 0.0)
    out_ref[...] = o
    rowv = lax.broadcasted_iota(jnp.int32, (BR, H), 0) + i * BR
    om = jnp.where(rowv < N_REAL, o, 0.0)
    st_ref[0, 0, :] = jnp.sum(om, axis=0)
    st_ref[0, 1, :] = jnp.sum(om * om, axis=0)


def _run_final_relu(partials, xw, dinv_col, b):
    return pl.pallas_call(
        _final_relu_body,
        grid=(NBLK,),
        in_specs=[pl.BlockSpec((2, BR, H), lambda i: (0, i, 0)),
                  pl.BlockSpec((BR, H), lambda i: (i, 0)),
                  pl.BlockSpec((BR, 1), lambda i: (i, 0)),
                  pl.BlockSpec((1, H), lambda i: (0, 0))],
        out_specs=[pl.BlockSpec((BR, H), lambda i: (i, 0)),
                   pl.BlockSpec((1, 2, H), lambda i: (i, 0, 0))],
        out_shape=[jax.ShapeDtypeStruct((NP, H), jnp.float32),
                   jax.ShapeDtypeStruct((NBLK, 2, H), jnp.float32)],
    )(partials, xw, dinv_col, b)


def _norm_proj_body(o2_ref, st_ref, gamma_ref, beta_ref, wp_ref, bp_ref,
                    out_ref):
    st = jnp.sum(st_ref[...], axis=0)
    cnt = jnp.float32(N_REAL)
    mean = st[0:1, :] / cnt
    var = st[1:2, :] / cnt - mean * mean
    o2 = o2_ref[...]
    o2n = (o2 - mean) / jnp.sqrt(var + 1e-5) * gamma_ref[...] + beta_ref[...]
    out_ref[...] = (lax.dot_general(o2n, wp_ref[...], (((1,), (1,)), ((), ())))
                    + bp_ref[...])


def _run_norm_proj(out2, stats, gamma, beta, wp, bp):
    return pl.pallas_call(
        _norm_proj_body,
        grid=(NBLK,),
        in_specs=[pl.BlockSpec((BR, H), lambda i: (i, 0)),
                  pl.BlockSpec((NBLK, 2, H), lambda i: (0, 0, 0)),
                  pl.BlockSpec((1, H), lambda i: (0, 0)),
                  pl.BlockSpec((1, H), lambda i: (0, 0)),
                  pl.BlockSpec((O_DIM, H), lambda i: (0, 0)),
                  pl.BlockSpec((1, O_DIM), lambda i: (0, 0))],
        out_specs=pl.BlockSpec((BR, O_DIM), lambda i: (i, 0)),
        out_shape=jax.ShapeDtypeStruct((NP, O_DIM), jnp.float32),
    )(out2, stats, gamma, beta, wp, bp)


def _split_gru_weights(wih, whh, bih, bhh):
    wir, wiz, win = jnp.split(wih, 3, axis=0)
    whr, whz, whn = jnp.split(whh, 3, axis=0)
    bir, biz, bin_ = jnp.split(bih, 3)
    bhr, bhz, bhn = jnp.split(bhh, 3)
    r2 = lambda a: a.reshape(1, H)
    return [wir, whr, r2(bir), r2(bhr), wiz, whz, r2(biz), r2(bhz),
            win, whn, r2(bin_), r2(bhn)]


@jax.jit
def kernel(x_seq, Wih_sim, Whh_sim, bih_sim, bhh_sim, Wih, Whh, bih, bhh,
           W1, b1, W2, b2, gamma, beta, Wp, bp):
    x0 = x_seq.reshape(4, F_IN, N_REAL)
    xp = jnp.pad(x0, ((0, 0), (0, 0), (0, NP - N_REAL)))

    wsets = (_split_gru_weights(Wih_sim, Whh_sim, bih_sim, bhh_sim)
             + _split_gru_weights(Wih, Whh, bih, bhh))
    h_sim, xw1 = _run_gru(xp, wsets, W1)

    vals, idx = _run_simtopk(h_sim)
    idx_flat = idx.reshape(NP * K)
    vals_flat = vals.reshape(NP * K)

    deg_p = _run_deg(idx_flat, vals_flat)
    dinv = _run_dinv(deg_p)
    dinv_flat = dinv.reshape(NP)
    dinv_col = dinv.reshape(NP, 1)

    p1 = _run_msg(xw1, idx_flat, vals_flat, dinv_flat)
    xw2 = _run_combine(p1, xw1, dinv_col, b1.reshape(1, H), W2)

    p2 = _run_msg(xw2, idx_flat, vals_flat, dinv_flat)
    out2, stats = _run_final_relu(p2, xw2, dinv_col, b2.reshape(1, H))

    out = _run_norm_proj(out2, stats, gamma.reshape(1, H),
                         beta.reshape(1, H), Wp, bp.reshape(1, O_DIM))
    return out[:N_REAL].reshape(1, N_REAL, O_DIM)


# speedup vs baseline: 28.8401x; 28.8401x over previous
import functools

import jax
import jax.numpy as jnp
from jax import lax
from jax.experimental import pallas as pl
from jax.experimental.pallas import tpu as pltpu
from jax.experimental.pallas import tpu_sc as plsc

N_REAL = 10000
NP = 10240
K = 8
F_IN = 128
H = 64
O_DIM = 32
BR = 256
NBLK = NP // BR
NW = 32
RPW = NP // NW
EPW = RPW * K
GPW = EPW // 16
TPS = 16
RPT = NP // TPS

_NEG = -3e38


def _gru_body(x_ref, *refs):
    (wir_s, whr_s, br_s, bhr_s, wiz_s, whz_s, bz_s, bhz_s, win_s, whn_s,
     bn_s, bhn_s,
     wir_o, whr_o, br_o, bhr_o, wiz_o, whz_o, bz_o, bhz_o, win_o, whn_o,
     bn_o, bhn_o,
     w1, hs_ref, xw1_ref, xw1t_ref) = refs

    def one_gru(wir, whr, bir, bhr, wiz, whz, biz, bhz, win, whn, bin_, bhn):
        h = jnp.zeros((BR, H), jnp.float32)
        for l in range(4):
            xl = x_ref[l]
            dn = (((0,), (1,)), ((), ()))
            gir = lax.dot_general(xl, wir[...], dn) + bir[...]
            giz = lax.dot_general(xl, wiz[...], dn) + biz[...]
            gin = lax.dot_general(xl, win[...], dn) + bin_[...]
            dnh = (((1,), (1,)), ((), ()))
            ghr = lax.dot_general(h, whr[...], dnh) + bhr[...]
            ghz = lax.dot_general(h, whz[...], dnh) + bhz[...]
            ghn = lax.dot_general(h, whn[...], dnh) + bhn[...]
            r = jax.nn.sigmoid(gir + ghr)
            z = jax.nn.sigmoid(giz + ghz)
            n = jnp.tanh(gin + r * ghn)
            h = (1.0 - z) * n + z * h
        return h

    h_sim = one_gru(wir_s, whr_s, br_s, bhr_s, wiz_s, whz_s, bz_s, bhz_s,
                    win_s, whn_s, bn_s, bhn_s)
    h_out = one_gru(wir_o, whr_o, br_o, bhr_o, wiz_o, whz_o, bz_o, bhz_o,
                    win_o, whn_o, bn_o, bhn_o)
    hs_ref[...] = h_sim
    xw1 = lax.dot_general(h_out, w1[...], (((1,), (1,)), ((), ())))
    xw1_ref[...] = xw1
    xw1t_ref[...] = jnp.transpose(xw1, (1, 0))


def _run_gru(xp, wsets, w1):
    full = lambda a: pl.BlockSpec(a.shape, lambda i: (0,) * a.ndim)
    in_specs = [pl.BlockSpec((4, F_IN, BR), lambda i: (0, 0, i))]
    in_specs += [full(a) for a in wsets] + [full(w1)]
    out_specs = [pl.BlockSpec((BR, H), lambda i: (i, 0)),
                 pl.BlockSpec((BR, H), lambda i: (i, 0)),
                 pl.BlockSpec((H, BR), lambda i: (0, i))]
    return pl.pallas_call(
        _gru_body,
        grid=(NBLK,),
        in_specs=in_specs,
        out_specs=out_specs,
        out_shape=[jax.ShapeDtypeStruct((NP, H), jnp.float32),
                   jax.ShapeDtypeStruct((NP, H), jnp.float32),
                   jax.ShapeDtypeStruct((H, NP), jnp.float32)],
    )(xp, *wsets, w1)


def _simtopk_body(hrow_ref, hall_ref, vals_ref, idx_ref, sbuf_ref):
    i = pl.program_id(0)
    hrow = hrow_ref[...]
    bc = 2048
    for c in range(NP // bc):
        hc = hall_ref[pl.ds(c * bc, bc), :]
        s = lax.dot_general(hrow, hc, (((1,), (1,)), ((), ())))
        col = lax.broadcasted_iota(jnp.int32, (BR, bc), 1) + c * bc
        row = lax.broadcasted_iota(jnp.int32, (BR, bc), 0) + i * BR
        s = jnp.where(col == row, -1e9, s)
        s = jnp.where(col >= N_REAL, _NEG, s)
        sbuf_ref[:, pl.ds(c * bc, bc)] = s

    coln = lax.broadcasted_iota(jnp.int32, (BR, NP), 1)
    lane8 = lax.broadcasted_iota(jnp.int32, (BR, K), 1)
    vals8 = jnp.zeros((BR, K), jnp.float32)
    idx8 = jnp.zeros((BR, K), jnp.int32)
    for t in range(K):
        sb = sbuf_ref[...]
        m = jnp.max(sb, axis=1, keepdims=True)
        ism = sb == m
        pos = jnp.min(jnp.where(ism, coln, 2 ** 30), axis=1,
                      keepdims=True)
        vals8 = jnp.where(lane8 == t, m, vals8)
        idx8 = jnp.where(lane8 == t, pos, idx8)
        if t < K - 1:
            sbuf_ref[...] = jnp.where(coln == pos, _NEG, sb)
    rowv = lax.broadcasted_iota(jnp.int32, (BR, K), 0) + i * BR
    valid = rowv < N_REAL
    vals_ref[...] = jnp.where(valid, vals8, 0.0)
    idx_ref[...] = jnp.where(valid, idx8, 0)


def _run_simtopk(h_sim):
    return pl.pallas_call(
        _simtopk_body,
        grid=(NBLK,),
        in_specs=[pl.BlockSpec((BR, H), lambda i: (i, 0)),
                  pl.BlockSpec((NP, H), lambda i: (0, 0))],
        out_specs=[pl.BlockSpec((BR, K), lambda i: (i, 0)),
                   pl.BlockSpec((BR, K), lambda i: (i, 0))],
        out_shape=[jax.ShapeDtypeStruct((NP, K), jnp.float32),
                   jax.ShapeDtypeStruct((NP, K), jnp.int32)],
        scratch_shapes=[pltpu.VMEM((BR, NP), jnp.float32)],
    )(h_sim, h_sim)


def _deg_body(idx_hbm, vals_hbm, out_hbm, idx_v, vals_v, deg_v):
    cid = lax.axis_index("c")
    sid = lax.axis_index("s")
    wid = cid * TPS + sid
    base = wid * EPW
    pltpu.sync_copy(idx_hbm.at[pl.ds(base, EPW)], idx_v)
    pltpu.sync_copy(vals_hbm.at[pl.ds(base, EPW)], vals_v)
    zero = jnp.zeros((16,), jnp.float32)

    def zb(j, carry):
        deg_v[pl.ds(j * 16, 16)] = zero
        return carry
    lax.fori_loop(0, 2 * NP // 16, zb, 0)

    lane = lax.iota(jnp.int32, 16)
    off = jnp.where(lane >= 8, NP, 0).astype(jnp.int32)

    def body(g, carry):
        d16 = idx_v[pl.ds(g * 16, 16)]
        w16 = vals_v[pl.ds(g * 16, 16)]
        plsc.addupdate_scatter(deg_v, [d16 + off], w16)
        return carry
    lax.fori_loop(0, GPW, body, 0)

    def fold(j, carry):
        a = deg_v[pl.ds(j * 16, 16)]
        b = deg_v[pl.ds(NP + j * 16, 16)]
        deg_v[pl.ds(j * 16, 16)] = a + b
        return carry
    lax.fori_loop(0, NP // 16, fold, 0)
    pltpu.sync_copy(deg_v.at[pl.ds(0, NP)], out_hbm.at[pl.ds(wid * NP, NP)])


def _run_deg(idx_flat, vals_flat):
    mesh = plsc.VectorSubcoreMesh(core_axis_name="c", subcore_axis_name="s")
    kfn = pl.kernel(
        _deg_body,
        out_type=jax.ShapeDtypeStruct((NW * NP,), jnp.float32),
        mesh=mesh,
        scratch_types=[pltpu.VMEM((EPW,), jnp.int32),
                       pltpu.VMEM((EPW,), jnp.float32),
                       pltpu.VMEM((2 * NP,), jnp.float32)],
        compiler_params=pltpu.CompilerParams(needs_layout_passes=False),
    )
    return kfn(idx_flat, vals_flat).reshape(NW, NP)


def _dinv_body(degp_ref, dinv_ref):
    j = pl.program_id(0)
    dsum = jnp.sum(degp_ref[...], axis=0, keepdims=True)
    col = lax.broadcasted_iota(jnp.int32, (1, 1024), 1) + j * 1024
    deg = dsum + jnp.where(col < N_REAL, 1.0, 0.0)
    dinv_ref[...] = jnp.where(deg > 0, lax.rsqrt(deg), 0.0)


def _run_dinv(deg_p):
    return pl.pallas_call(
        _dinv_body,
        grid=(NP // 1024,),
        in_specs=[pl.BlockSpec((NW, 1024), lambda j: (0, j))],
        out_specs=pl.BlockSpec((1, 1024), lambda j: (0, j)),
        out_shape=jax.ShapeDtypeStruct((1, NP), jnp.float32),
    )(deg_p)


FPT = H // NW
ECH = 8192
NWIN = NP * K // ECH


def _msg_body(xwt_hbm, idx_hbm, vals_hbm, dinv_hbm, out_hbm,
              xt_v, idx_v, vals_v, dinv_v, acc_v):
    cid = lax.axis_index("c")
    sid = lax.axis_index("s")
    wid = cid * TPS + sid
    fbase = wid * FPT
    pltpu.sync_copy(dinv_hbm, dinv_v)
    for f in range(FPT):
        pltpu.sync_copy(xwt_hbm.at[pl.ds((fbase + f) * NP, NP)],
                        xt_v.at[pl.ds(f * NP, NP)])

    zero = jnp.zeros((16,), jnp.float32)

    def zb(j, carry):
        acc_v[pl.ds(j * 16, 16)] = zero
        return carry
    lax.fori_loop(0, FPT * 2 * NP // 16, zb, 0)

    lane = lax.iota(jnp.int32, 16)
    off = jnp.where(lane >= 8, NP, 0).astype(jnp.int32)

    def win(w, carry):
        pltpu.sync_copy(idx_hbm.at[pl.ds(w * ECH, ECH)], idx_v)
        pltpu.sync_copy(vals_hbm.at[pl.ds(w * ECH, ECH)], vals_v)

        def body(g, carry2):
            e0 = g * 16
            d16 = idx_v[pl.ds(e0, 16)]
            w16 = vals_v[pl.ds(e0, 16)]
            src = lax.shift_right_logical(w * ECH + e0 + lane, 3)
            dsrc = plsc.load_gather(dinv_v, [src])
            ddst = plsc.load_gather(dinv_v, [d16])
            scale = dsrc * w16 * ddst
            tgt = d16 + off
            for f in range(FPT):
                xg = plsc.load_gather(xt_v, [src + f * NP])
                plsc.addupdate_scatter(acc_v, [tgt + f * 2 * NP],
                                       scale * xg)
            return carry2
        lax.fori_loop(0, ECH // 16, body, 0)
        return carry
    lax.fori_loop(0, NWIN, win, 0)

    def fold(j, carry):
        for f in range(FPT):
            a = acc_v[pl.ds(f * 2 * NP + j * 16, 16)]
            b = acc_v[pl.ds(f * 2 * NP + NP + j * 16, 16)]
            acc_v[pl.ds(f * 2 * NP + j * 16, 16)] = a + b
        return carry
    lax.fori_loop(0, NP // 16, fold, 0)
    for f in range(FPT):
        pltpu.sync_copy(acc_v.at[pl.ds(f * 2 * NP, NP)],
                        out_hbm.at[pl.ds((fbase + f) * NP, NP)])


def _run_msg(xwt_flat, idx_flat, vals_flat, dinv):
    mesh = plsc.VectorSubcoreMesh(core_axis_name="c", subcore_axis_name="s")
    kfn = pl.kernel(
        _msg_body,
        out_type=jax.ShapeDtypeStruct((H * NP,), jnp.float32),
        mesh=mesh,
        scratch_types=[pltpu.VMEM((FPT * NP,), jnp.float32),
                       pltpu.VMEM((ECH,), jnp.int32),
                       pltpu.VMEM((ECH,), jnp.float32),
                       pltpu.VMEM((NP,), jnp.float32),
                       pltpu.VMEM((FPT * 2 * NP,), jnp.float32)],
        compiler_params=pltpu.CompilerParams(needs_layout_passes=False),
    )
    return kfn(xwt_flat, idx_flat, vals_flat, dinv).reshape(H, NP)


def _combine_body(p_ref, xw_ref, dinv_ref, b_ref, w_ref, out_ref, outt_ref):
    p = jnp.transpose(p_ref[...], (1, 0))
    dv = dinv_ref[...]
    o = p + dv * dv * xw_ref[...] + b_ref[...]
    o = jnp.maximum(o, 0.0)
    xw2 = lax.dot_general(o, w_ref[...], (((1,), (1,)), ((), ())))
    out_ref[...] = xw2
    outt_ref[...] = jnp.transpose(xw2, (1, 0))


def _run_combine(agg_t, xw, dinv_col, b, w):
    return pl.pallas_call(
        _combine_body,
        grid=(NBLK,),
        in_specs=[pl.BlockSpec((H, BR), lambda i: (0, i)),
                  pl.BlockSpec((BR, H), lambda i: (i, 0)),
                  pl.BlockSpec((BR, 1), lambda i: (i, 0)),
                  pl.BlockSpec((1, H), lambda i: (0, 0)),
                  pl.BlockSpec(w.shape, lambda i: (0, 0))],
        out_specs=[pl.BlockSpec((BR, H), lambda i: (i, 0)),
                   pl.BlockSpec((H, BR), lambda i: (0, i))],
        out_shape=[jax.ShapeDtypeStruct((NP, H), jnp.float32),
                   jax.ShapeDtypeStruct((H, NP), jnp.float32)],
    )(agg_t, xw, dinv_col, b, w)


def _final_relu_body(p_ref, xw_ref, dinv_ref, b_ref, out_ref, st_ref):
    i = pl.program_id(0)
    p = jnp.transpose(p_ref[...], (1, 0))
    dv = dinv_ref[...]
    o = p + dv * dv * xw_ref[...] + b_ref[...]
    o = jnp.maximum(o, 0.0)
    out_ref[...] = o
    rowv = lax.broadcasted_iota(jnp.int32, (BR, H), 0) + i * BR
    om = jnp.where(rowv < N_REAL, o, 0.0)
    st_ref[0, 0, :] = jnp.sum(om, axis=0)
    st_ref[0, 1, :] = jnp.sum(om * om, axis=0)


def _run_final_relu(agg_t, xw, dinv_col, b):
    return pl.pallas_call(
        _final_relu_body,
        grid=(NBLK,),
        in_specs=[pl.BlockSpec((H, BR), lambda i: (0, i)),
                  pl.BlockSpec((BR, H), lambda i: (i, 0)),
                  pl.BlockSpec((BR, 1), lambda i: (i, 0)),
                  pl.BlockSpec((1, H), lambda i: (0, 0))],
        out_specs=[pl.BlockSpec((BR, H), lambda i: (i, 0)),
                   pl.BlockSpec((1, 2, H), lambda i: (i, 0, 0))],
        out_shape=[jax.ShapeDtypeStruct((NP, H), jnp.float32),
                   jax.ShapeDtypeStruct((NBLK, 2, H), jnp.float32)],
    )(agg_t, xw, dinv_col, b)


def _norm_proj_body(o2_ref, st_ref, gamma_ref, beta_ref, wp_ref, bp_ref,
                    out_ref):
    st = jnp.sum(st_ref[...], axis=0)
    cnt = jnp.float32(N_REAL)
    mean = st[0:1, :] / cnt
    var = st[1:2, :] / cnt - mean * mean
    o2 = o2_ref[...]
    o2n = (o2 - mean) / jnp.sqrt(var + 1e-5) * gamma_ref[...] + beta_ref[...]
    out_ref[...] = (lax.dot_general(o2n, wp_ref[...], (((1,), (1,)), ((), ())))
                    + bp_ref[...])


def _run_norm_proj(out2, stats, gamma, beta, wp, bp):
    return pl.pallas_call(
        _norm_proj_body,
        grid=(NBLK,),
        in_specs=[pl.BlockSpec((BR, H), lambda i: (i, 0)),
                  pl.BlockSpec((NBLK, 2, H), lambda i: (0, 0, 0)),
                  pl.BlockSpec((1, H), lambda i: (0, 0)),
                  pl.BlockSpec((1, H), lambda i: (0, 0)),
                  pl.BlockSpec((O_DIM, H), lambda i: (0, 0)),
                  pl.BlockSpec((1, O_DIM), lambda i: (0, 0))],
        out_specs=pl.BlockSpec((BR, O_DIM), lambda i: (i, 0)),
        out_shape=jax.ShapeDtypeStruct((NP, O_DIM), jnp.float32),
    )(out2, stats, gamma, beta, wp, bp)


def _split_gru_weights(wih, whh, bih, bhh):
    wir, wiz, win = jnp.split(wih, 3, axis=0)
    whr, whz, whn = jnp.split(whh, 3, axis=0)
    bir, biz, bin_ = jnp.split(bih, 3)
    bhr, bhz, bhn = jnp.split(bhh, 3)
    r2 = lambda a: a.reshape(1, H)
    return [wir, whr, r2(bir), r2(bhr), wiz, whz, r2(biz), r2(bhz),
            win, whn, r2(bin_), r2(bhn)]


@jax.jit
def kernel(x_seq, Wih_sim, Whh_sim, bih_sim, bhh_sim, Wih, Whh, bih, bhh,
           W1, b1, W2, b2, gamma, beta, Wp, bp):
    x0 = x_seq.reshape(4, F_IN, N_REAL)
    xp = jnp.pad(x0, ((0, 0), (0, 0), (0, NP - N_REAL)))

    wsets = (_split_gru_weights(Wih_sim, Whh_sim, bih_sim, bhh_sim)
             + _split_gru_weights(Wih, Whh, bih, bhh))
    h_sim, xw1, xw1_t = _run_gru(xp, wsets, W1)

    vals, idx = _run_simtopk(h_sim)
    idx_flat = idx.reshape(NP * K)
    vals_flat = vals.reshape(NP * K)

    deg_p = _run_deg(idx_flat, vals_flat)
    dinv = _run_dinv(deg_p)
    dinv_flat = dinv.reshape(NP)
    dinv_col = dinv.reshape(NP, 1)

    p1 = _run_msg(xw1_t.reshape(H * NP), idx_flat, vals_flat, dinv_flat)
    xw2, xw2_t = _run_combine(p1, xw1, dinv_col, b1.reshape(1, H), W2)

    p2 = _run_msg(xw2_t.reshape(H * NP), idx_flat, vals_flat, dinv_flat)
    out2, stats = _run_final_relu(p2, xw2, dinv_col, b2.reshape(1, H))

    out = _run_norm_proj(out2, stats, gamma.reshape(1, H),
                         beta.reshape(1, H), Wp, bp.reshape(1, O_DIM))
    return out[:N_REAL].reshape(1, N_REAL, O_DIM)


# speedup vs baseline: 35.2895x; 1.2236x over previous
import functools

import jax
import jax.numpy as jnp
from jax import lax
from jax.experimental import pallas as pl
from jax.experimental.pallas import tpu as pltpu
from jax.experimental.pallas import tpu_sc as plsc

N_REAL = 10000
NP = 10240
K = 8
F_IN = 128
H = 64
O_DIM = 32
BR = 256
NBLK = NP // BR
NW = 32
RPW = NP // NW
EPW = RPW * K
GPW = EPW // 16
TPS = 16
RPT = NP // TPS

_NEG = -3e38


def _gru_body(x_ref, *refs):
    (wir_s, whr_s, br_s, bhr_s, wiz_s, whz_s, bz_s, bhz_s, win_s, whn_s,
     bn_s, bhn_s,
     wir_o, whr_o, br_o, bhr_o, wiz_o, whz_o, bz_o, bhz_o, win_o, whn_o,
     bn_o, bhn_o,
     w1, hs_ref, xw1_ref, xw1t_ref) = refs

    def one_gru(wir, whr, bir, bhr, wiz, whz, biz, bhz, win, whn, bin_, bhn):
        h = jnp.zeros((BR, H), jnp.float32)
        for l in range(4):
            xl = x_ref[l]
            dn = (((0,), (1,)), ((), ()))
            gir = lax.dot_general(xl, wir[...], dn) + bir[...]
            giz = lax.dot_general(xl, wiz[...], dn) + biz[...]
            gin = lax.dot_general(xl, win[...], dn) + bin_[...]
            dnh = (((1,), (1,)), ((), ()))
            ghr = lax.dot_general(h, whr[...], dnh) + bhr[...]
            ghz = lax.dot_general(h, whz[...], dnh) + bhz[...]
            ghn = lax.dot_general(h, whn[...], dnh) + bhn[...]
            r = jax.nn.sigmoid(gir + ghr)
            z = jax.nn.sigmoid(giz + ghz)
            n = jnp.tanh(gin + r * ghn)
            h = (1.0 - z) * n + z * h
        return h

    h_sim = one_gru(wir_s, whr_s, br_s, bhr_s, wiz_s, whz_s, bz_s, bhz_s,
                    win_s, whn_s, bn_s, bhn_s)
    h_out = one_gru(wir_o, whr_o, br_o, bhr_o, wiz_o, whz_o, bz_o, bhz_o,
                    win_o, whn_o, bn_o, bhn_o)
    hs_ref[...] = h_sim
    xw1 = lax.dot_general(h_out, w1[...], (((1,), (1,)), ((), ())))
    xw1_ref[...] = xw1
    xw1t_ref[...] = jnp.transpose(xw1, (1, 0))


def _run_gru(xp, wsets, w1):
    full = lambda a: pl.BlockSpec(a.shape, lambda i: (0,) * a.ndim)
    in_specs = [pl.BlockSpec((4, F_IN, BR), lambda i: (0, 0, i))]
    in_specs += [full(a) for a in wsets] + [full(w1)]
    out_specs = [pl.BlockSpec((BR, H), lambda i: (i, 0)),
                 pl.BlockSpec((BR, H), lambda i: (i, 0)),
                 pl.BlockSpec((H, BR), lambda i: (0, i))]
    return pl.pallas_call(
        _gru_body,
        grid=(NBLK,),
        in_specs=in_specs,
        out_specs=out_specs,
        out_shape=[jax.ShapeDtypeStruct((NP, H), jnp.float32),
                   jax.ShapeDtypeStruct((NP, H), jnp.float32),
                   jax.ShapeDtypeStruct((H, NP), jnp.float32)],
    )(xp, *wsets, w1)


NCHUNK = NP // 128


def _simtopk_body(hrow_ref, hall_ref, vals_ref, idx_ref,
                  sbuf_ref, cpos_ref, cand_ref):
    i = pl.program_id(0)
    hrow = hrow_ref[...]
    bc = 2048
    cm_parts = []
    for c in range(NP // bc):
        hc = hall_ref[pl.ds(c * bc, bc), :]
        s = lax.dot_general(hrow, hc, (((1,), (1,)), ((), ())))
        col = lax.broadcasted_iota(jnp.int32, (BR, bc), 1) + c * bc
        row = lax.broadcasted_iota(jnp.int32, (BR, bc), 0) + i * BR
        s = jnp.where(col == row, -1e9, s)
        s = jnp.where(col >= N_REAL, _NEG, s)
        sbuf_ref[:, pl.ds(c * bc, bc)] = s
        cm_parts.append(jnp.max(s.reshape(BR, bc // 128, 128), axis=2))

    cm = jnp.concatenate(cm_parts, axis=1)
    ckidx = lax.broadcasted_iota(jnp.int32, (BR, NCHUNK), 1)
    lane8 = lax.broadcasted_iota(jnp.int32, (BR, K), 1)
    cpos8 = jnp.zeros((BR, K), jnp.int32)
    for t in range(K):
        m = jnp.max(cm, axis=1, keepdims=True)
        cp = jnp.min(jnp.where(cm == m, ckidx, 2 ** 30), axis=1,
                     keepdims=True)
        cpos8 = jnp.where(lane8 == t, cp, cpos8)
        if t < K - 1:
            cm = jnp.where(ckidx == cp, _NEG, cm)
    cpos_ref[...] = cpos8

    sub8 = lax.broadcasted_iota(jnp.int32, (8, 128), 0)

    def gath(g, carry):
        rbase = pl.multiple_of(g * 8, 8)
        for j in range(K):
            acc = jnp.zeros((8, 128), jnp.float32)
            for r in range(8):
                c = cpos_ref[g * 8 + r, j]
                coff = pl.multiple_of(c * 128, 128)
                blk = sbuf_ref[pl.ds(rbase, 8), pl.ds(coff, 128)]
                acc = jnp.where(sub8 == r, blk, acc)
            cand_ref[pl.ds(rbase, 8), pl.ds(j * 128, 128)] = acc
        return carry
    lax.fori_loop(0, BR // 8, gath, 0)

    vals8 = jnp.zeros((BR, K), jnp.float32)
    idx8 = jnp.zeros((BR, K), jnp.int32)
    gidx = (jnp.broadcast_to(cpos8[:, :, None], (BR, K, 128)) * 128
            + lax.broadcasted_iota(jnp.int32, (BR, K, 128), 2)
            ).reshape(BR, K * 128)
    cand = cand_ref[...]
    for t in range(K):
        m = jnp.max(cand, axis=1, keepdims=True)
        pos = jnp.min(jnp.where(cand == m, gidx, 2 ** 30), axis=1,
                      keepdims=True)
        vals8 = jnp.where(lane8 == t, m, vals8)
        idx8 = jnp.where(lane8 == t, pos, idx8)
        if t < K - 1:
            cand = jnp.where(gidx == pos, _NEG, cand)
    rowv = lax.broadcasted_iota(jnp.int32, (BR, K), 0) + i * BR
    valid = rowv < N_REAL
    vals_ref[...] = jnp.where(valid, vals8, 0.0)
    idx_ref[...] = jnp.where(valid, idx8, 0)


def _run_simtopk(h_sim):
    return pl.pallas_call(
        _simtopk_body,
        grid=(NBLK,),
        in_specs=[pl.BlockSpec((BR, H), lambda i: (i, 0)),
                  pl.BlockSpec((NP, H), lambda i: (0, 0))],
        out_specs=[pl.BlockSpec((BR, K), lambda i: (i, 0)),
                   pl.BlockSpec((BR, K), lambda i: (i, 0))],
        out_shape=[jax.ShapeDtypeStruct((NP, K), jnp.float32),
                   jax.ShapeDtypeStruct((NP, K), jnp.int32)],
        scratch_shapes=[pltpu.VMEM((BR, NP), jnp.float32),
                        pltpu.VMEM((BR, K), jnp.int32),
                        pltpu.VMEM((BR, K * 128), jnp.float32)],
    )(h_sim, h_sim)


def _deg_body(idx_hbm, vals_hbm, out_hbm, idx_v, vals_v, deg_v):
    cid = lax.axis_index("c")
    sid = lax.axis_index("s")
    wid = cid * TPS + sid
    base = wid * EPW
    pltpu.sync_copy(idx_hbm.at[pl.ds(base, EPW)], idx_v)
    pltpu.sync_copy(vals_hbm.at[pl.ds(base, EPW)], vals_v)
    zero = jnp.zeros((16,), jnp.float32)

    def zb(j, carry):
        deg_v[pl.ds(j * 16, 16)] = zero
        return carry
    lax.fori_loop(0, 2 * NP // 16, zb, 0)

    lane = lax.iota(jnp.int32, 16)
    off = jnp.where(lane >= 8, NP, 0).astype(jnp.int32)

    def body(g, carry):
        d16 = idx_v[pl.ds(g * 16, 16)]
        w16 = vals_v[pl.ds(g * 16, 16)]
        plsc.addupdate_scatter(deg_v, [d16 + off], w16)
        return carry
    lax.fori_loop(0, GPW, body, 0)

    def fold(j, carry):
        a = deg_v[pl.ds(j * 16, 16)]
        b = deg_v[pl.ds(NP + j * 16, 16)]
        deg_v[pl.ds(j * 16, 16)] = a + b
        return carry
    lax.fori_loop(0, NP // 16, fold, 0)
    pltpu.sync_copy(deg_v.at[pl.ds(0, NP)], out_hbm.at[pl.ds(wid * NP, NP)])


def _run_deg(idx_flat, vals_flat):
    mesh = plsc.VectorSubcoreMesh(core_axis_name="c", subcore_axis_name="s")
    kfn = pl.kernel(
        _deg_body,
        out_type=jax.ShapeDtypeStruct((NW * NP,), jnp.float32),
        mesh=mesh,
        scratch_types=[pltpu.VMEM((EPW,), jnp.int32),
                       pltpu.VMEM((EPW,), jnp.float32),
                       pltpu.VMEM((2 * NP,), jnp.float32)],
        compiler_params=pltpu.CompilerParams(needs_layout_passes=False),
    )
    return kfn(idx_flat, vals_flat).reshape(NW, NP)


def _dinv_body(degp_ref, dinv_ref):
    j = pl.program_id(0)
    dsum = jnp.sum(degp_ref[...], axis=0, keepdims=True)
    col = lax.broadcasted_iota(jnp.int32, (1, 1024), 1) + j * 1024
    deg = dsum + jnp.where(col < N_REAL, 1.0, 0.0)
    dinv_ref[...] = jnp.where(deg > 0, lax.rsqrt(deg), 0.0)


def _run_dinv(deg_p):
    return pl.pallas_call(
        _dinv_body,
        grid=(NP // 1024,),
        in_specs=[pl.BlockSpec((NW, 1024), lambda j: (0, j))],
        out_specs=pl.BlockSpec((1, 1024), lambda j: (0, j)),
        out_shape=jax.ShapeDtypeStruct((1, NP), jnp.float32),
    )(deg_p)


FPT = H // NW
ECH = 8192
NWIN = NP * K // ECH


def _msg_body(xwt_hbm, idx_hbm, vals_hbm, dinv_hbm, out_hbm,
              xt_v, idx_v, vals_v, dinv_v, acc_v):
    cid = lax.axis_index("c")
    sid = lax.axis_index("s")
    wid = cid * TPS + sid
    fbase = wid * FPT
    pltpu.sync_copy(dinv_hbm, dinv_v)
    for f in range(FPT):
        pltpu.sync_copy(xwt_hbm.at[pl.ds((fbase + f) * NP, NP)],
                        xt_v.at[pl.ds(f * NP, NP)])

    zero = jnp.zeros((16,), jnp.float32)

    def zb(j, carry):
        acc_v[pl.ds(j * 16, 16)] = zero
        return carry
    lax.fori_loop(0, FPT * 2 * NP // 16, zb, 0)

    lane = lax.iota(jnp.int32, 16)
    off = jnp.where(lane >= 8, NP, 0).astype(jnp.int32)

    def win(w, carry):
        pltpu.sync_copy(idx_hbm.at[pl.ds(w * ECH, ECH)], idx_v)
        pltpu.sync_copy(vals_hbm.at[pl.ds(w * ECH, ECH)], vals_v)

        def body(g, carry2):
            e0 = g * 16
            d16 = idx_v[pl.ds(e0, 16)]
            w16 = vals_v[pl.ds(e0, 16)]
            src = lax.shift_right_logical(w * ECH + e0 + lane, 3)
            dsrc = plsc.load_gather(dinv_v, [src])
            ddst = plsc.load_gather(dinv_v, [d16])
            scale = dsrc * w16 * ddst
            tgt = d16 + off
            for f in range(FPT):
                xg = plsc.load_gather(xt_v, [src + f * NP])
                plsc.addupdate_scatter(acc_v, [tgt + f * 2 * NP],
                                       scale * xg)
            return carry2
        lax.fori_loop(0, ECH // 16, body, 0)
        return carry
    lax.fori_loop(0, NWIN, win, 0)

    def fold(j, carry):
        for f in range(FPT):
            a = acc_v[pl.ds(f * 2 * NP + j * 16, 16)]
            b = acc_v[pl.ds(f * 2 * NP + NP + j * 16, 16)]
            acc_v[pl.ds(f * 2 * NP + j * 16, 16)] = a + b
        return carry
    lax.fori_loop(0, NP // 16, fold, 0)
    for f in range(FPT):
        pltpu.sync_copy(acc_v.at[pl.ds(f * 2 * NP, NP)],
                        out_hbm.at[pl.ds((fbase + f) * NP, NP)])


def _run_msg(xwt_flat, idx_flat, vals_flat, dinv):
    mesh = plsc.VectorSubcoreMesh(core_axis_name="c", subcore_axis_name="s")
    kfn = pl.kernel(
        _msg_body,
        out_type=jax.ShapeDtypeStruct((H * NP,), jnp.float32),
        mesh=mesh,
        scratch_types=[pltpu.VMEM((FPT * NP,), jnp.float32),
                       pltpu.VMEM((ECH,), jnp.int32),
                       pltpu.VMEM((ECH,), jnp.float32),
                       pltpu.VMEM((NP,), jnp.float32),
                       pltpu.VMEM((FPT * 2 * NP,), jnp.float32)],
        compiler_params=pltpu.CompilerParams(needs_layout_passes=False),
    )
    return kfn(xwt_flat, idx_flat, vals_flat, dinv).reshape(H, NP)


def _combine_body(p_ref, xw_ref, dinv_ref, b_ref, w_ref, out_ref, outt_ref):
    p = jnp.transpose(p_ref[...], (1, 0))
    dv = dinv_ref[...]
    o = p + dv * dv * xw_ref[...] + b_ref[...]
    o = jnp.maximum(o, 0.0)
    xw2 = lax.dot_general(o, w_ref[...], (((1,), (1,)), ((), ())))
    out_ref[...] = xw2
    outt_ref[...] = jnp.transpose(xw2, (1, 0))


def _run_combine(agg_t, xw, dinv_col, b, w):
    return pl.pallas_call(
        _combine_body,
        grid=(NBLK,),
        in_specs=[pl.BlockSpec((H, BR), lambda i: (0, i)),
                  pl.BlockSpec((BR, H), lambda i: (i, 0)),
                  pl.BlockSpec((BR, 1), lambda i: (i, 0)),
                  pl.BlockSpec((1, H), lambda i: (0, 0)),
                  pl.BlockSpec(w.shape, lambda i: (0, 0))],
        out_specs=[pl.BlockSpec((BR, H), lambda i: (i, 0)),
                   pl.BlockSpec((H, BR), lambda i: (0, i))],
        out_shape=[jax.ShapeDtypeStruct((NP, H), jnp.float32),
                   jax.ShapeDtypeStruct((H, NP), jnp.float32)],
    )(agg_t, xw, dinv_col, b, w)


def _final_relu_body(p_ref, xw_ref, dinv_ref, b_ref, out_ref, st_ref):
    i = pl.program_id(0)
    p = jnp.transpose(p_ref[...], (1, 0))
    dv = dinv_ref[...]
    o = p + dv * dv * xw_ref[...] + b_ref[...]
    o = jnp.maximum(o, 0.0)
    out_ref[...] = o
    rowv = lax.broadcasted_iota(jnp.int32, (BR, H), 0) + i * BR
    om = jnp.where(rowv < N_REAL, o, 0.0)
    st_ref[0, 0, :] = jnp.sum(om, axis=0)
    st_ref[0, 1, :] = jnp.sum(om * om, axis=0)


def _run_final_relu(agg_t, xw, dinv_col, b):
    return pl.pallas_call(
        _final_relu_body,
        grid=(NBLK,),
        in_specs=[pl.BlockSpec((H, BR), lambda i: (0, i)),
                  pl.BlockSpec((BR, H), lambda i: (i, 0)),
                  pl.BlockSpec((BR, 1), lambda i: (i, 0)),
                  pl.BlockSpec((1, H), lambda i: (0, 0))],
        out_specs=[pl.BlockSpec((BR, H), lambda i: (i, 0)),
                   pl.BlockSpec((1, 2, H), lambda i: (i, 0, 0))],
        out_shape=[jax.ShapeDtypeStruct((NP, H), jnp.float32),
                   jax.ShapeDtypeStruct((NBLK, 2, H), jnp.float32)],
    )(agg_t, xw, dinv_col, b)


def _norm_proj_body(o2_ref, st_ref, gamma_ref, beta_ref, wp_ref, bp_ref,
                    out_ref):
    st = jnp.sum(st_ref[...], axis=0)
    cnt = jnp.float32(N_REAL)
    mean = st[0:1, :] / cnt
    var = st[1:2, :] / cnt - mean * mean
    o2 = o2_ref[...]
    o2n = (o2 - mean) / jnp.sqrt(var + 1e-5) * gamma_ref[...] + beta_ref[...]
    out_ref[...] = (lax.dot_general(o2n, wp_ref[...], (((1,), (1,)), ((), ())))
                    + bp_ref[...])


def _run_norm_proj(out2, stats, gamma, beta, wp, bp):
    return pl.pallas_call(
        _norm_proj_body,
        grid=(NBLK,),
        in_specs=[pl.BlockSpec((BR, H), lambda i: (i, 0)),
                  pl.BlockSpec((NBLK, 2, H), lambda i: (0, 0, 0)),
                  pl.BlockSpec((1, H), lambda i: (0, 0)),
                  pl.BlockSpec((1, H), lambda i: (0, 0)),
                  pl.BlockSpec((O_DIM, H), lambda i: (0, 0)),
                  pl.BlockSpec((1, O_DIM), lambda i: (0, 0))],
        out_specs=pl.BlockSpec((BR, O_DIM), lambda i: (i, 0)),
        out_shape=jax.ShapeDtypeStruct((NP, O_DIM), jnp.float32),
    )(out2, stats, gamma, beta, wp, bp)


def _split_gru_weights(wih, whh, bih, bhh):
    wir, wiz, win = jnp.split(wih, 3, axis=0)
    whr, whz, whn = jnp.split(whh, 3, axis=0)
    bir, biz, bin_ = jnp.split(bih, 3)
    bhr, bhz, bhn = jnp.split(bhh, 3)
    r2 = lambda a: a.reshape(1, H)
    return [wir, whr, r2(bir), r2(bhr), wiz, whz, r2(biz), r2(bhz),
            win, whn, r2(bin_), r2(bhn)]


@jax.jit
def kernel(x_seq, Wih_sim, Whh_sim, bih_sim, bhh_sim, Wih, Whh, bih, bhh,
           W1, b1, W2, b2, gamma, beta, Wp, bp):
    x0 = x_seq.reshape(4, F_IN, N_REAL)
    xp = jnp.pad(x0, ((0, 0), (0, 0), (0, NP - N_REAL)))

    wsets = (_split_gru_weights(Wih_sim, Whh_sim, bih_sim, bhh_sim)
             + _split_gru_weights(Wih, Whh, bih, bhh))
    h_sim, xw1, xw1_t = _run_gru(xp, wsets, W1)

    vals, idx = _run_simtopk(h_sim)
    idx_flat = idx.reshape(NP * K)
    vals_flat = vals.reshape(NP * K)

    deg_p = _run_deg(idx_flat, vals_flat)
    dinv = _run_dinv(deg_p)
    dinv_flat = dinv.reshape(NP)
    dinv_col = dinv.reshape(NP, 1)

    p1 = _run_msg(xw1_t.reshape(H * NP), idx_flat, vals_flat, dinv_flat)
    xw2, xw2_t = _run_combine(p1, xw1, dinv_col, b1.reshape(1, H), W2)

    p2 = _run_msg(xw2_t.reshape(H * NP), idx_flat, vals_flat, dinv_flat)
    out2, stats = _run_final_relu(p2, xw2, dinv_col, b2.reshape(1, H))

    out = _run_norm_proj(out2, stats, gamma.reshape(1, H),
                         beta.reshape(1, H), Wp, bp.reshape(1, O_DIM))
    return out[:N_REAL].reshape(1, N_REAL, O_DIM)


# speedup vs baseline: 35.3539x; 1.0018x over previous
import functools

import jax
import jax.numpy as jnp
from jax import lax
from jax.experimental import pallas as pl
from jax.experimental.pallas import tpu as pltpu
from jax.experimental.pallas import tpu_sc as plsc

N_REAL = 10000
NP = 10240
K = 8
F_IN = 128
H = 64
O_DIM = 32
BR = 256
NBLK = NP // BR
NW = 32
RPW = NP // NW
EPW = RPW * K
GPW = EPW // 16
TPS = 16
RPT = NP // TPS

_NEG = -3e38


def _gru_body(x_ref, *refs):
    (wir_s, whr_s, br_s, bhr_s, wiz_s, whz_s, bz_s, bhz_s, win_s, whn_s,
     bn_s, bhn_s,
     wir_o, whr_o, br_o, bhr_o, wiz_o, whz_o, bz_o, bhz_o, win_o, whn_o,
     bn_o, bhn_o,
     w1, hs_ref, xw1_ref, xw1t_ref) = refs

    def one_gru(wir, whr, bir, bhr, wiz, whz, biz, bhz, win, whn, bin_, bhn):
        h = jnp.zeros((BR, H), jnp.float32)
        for l in range(4):
            xl = x_ref[l]
            dn = (((0,), (1,)), ((), ()))
            gir = lax.dot_general(xl, wir[...], dn) + bir[...]
            giz = lax.dot_general(xl, wiz[...], dn) + biz[...]
            gin = lax.dot_general(xl, win[...], dn) + bin_[...]
            dnh = (((1,), (1,)), ((), ()))
            ghr = lax.dot_general(h, whr[...], dnh) + bhr[...]
            ghz = lax.dot_general(h, whz[...], dnh) + bhz[...]
            ghn = lax.dot_general(h, whn[...], dnh) + bhn[...]
            r = jax.nn.sigmoid(gir + ghr)
            z = jax.nn.sigmoid(giz + ghz)
            n = jnp.tanh(gin + r * ghn)
            h = (1.0 - z) * n + z * h
        return h

    h_sim = one_gru(wir_s, whr_s, br_s, bhr_s, wiz_s, whz_s, bz_s, bhz_s,
                    win_s, whn_s, bn_s, bhn_s)
    h_out = one_gru(wir_o, whr_o, br_o, bhr_o, wiz_o, whz_o, bz_o, bhz_o,
                    win_o, whn_o, bn_o, bhn_o)
    hs_ref[...] = h_sim
    xw1 = lax.dot_general(h_out, w1[...], (((1,), (1,)), ((), ())))
    xw1_ref[...] = xw1
    xw1t_ref[...] = jnp.transpose(xw1, (1, 0))


def _run_gru(xp, wsets, w1):
    full = lambda a: pl.BlockSpec(a.shape, lambda i: (0,) * a.ndim)
    in_specs = [pl.BlockSpec((4, F_IN, BR), lambda i: (0, 0, i))]
    in_specs += [full(a) for a in wsets] + [full(w1)]
    out_specs = [pl.BlockSpec((BR, H), lambda i: (i, 0)),
                 pl.BlockSpec((BR, H), lambda i: (i, 0)),
                 pl.BlockSpec((H, BR), lambda i: (0, i))]
    return pl.pallas_call(
        _gru_body,
        grid=(NBLK,),
        in_specs=in_specs,
        out_specs=out_specs,
        out_shape=[jax.ShapeDtypeStruct((NP, H), jnp.float32),
                   jax.ShapeDtypeStruct((NP, H), jnp.float32),
                   jax.ShapeDtypeStruct((H, NP), jnp.float32)],
    )(xp, *wsets, w1)


NCHUNK = NP // 128


def _simtopk_body(hrow_ref, hall_ref, vals_ref, idx_ref,
                  sbuf_ref, cpos_ref, cand_ref):
    i = pl.program_id(0)
    hrow = hrow_ref[...]
    bc = 2048
    cm_parts = []
    for c in range(NP // bc):
        hc = hall_ref[pl.ds(c * bc, bc), :]
        s = lax.dot_general(hrow, hc, (((1,), (1,)), ((), ())))
        col = lax.broadcasted_iota(jnp.int32, (BR, bc), 1) + c * bc
        row = lax.broadcasted_iota(jnp.int32, (BR, bc), 0) + i * BR
        s = jnp.where(col == row, -1e9, s)
        s = jnp.where(col >= N_REAL, _NEG, s)
        sbuf_ref[:, pl.ds(c * bc, bc)] = s
        cm_parts.append(jnp.max(s.reshape(BR, bc // 128, 128), axis=2))

    cm = jnp.concatenate(cm_parts, axis=1)
    ckidx = lax.broadcasted_iota(jnp.int32, (BR, NCHUNK), 1)
    lane8 = lax.broadcasted_iota(jnp.int32, (BR, K), 1)
    cpos8 = jnp.zeros((BR, K), jnp.int32)
    for t in range(K):
        m = jnp.max(cm, axis=1, keepdims=True)
        cp = jnp.min(jnp.where(cm == m, ckidx, 2 ** 30), axis=1,
                     keepdims=True)
        cpos8 = jnp.where(lane8 == t, cp, cpos8)
        if t < K - 1:
            cm = jnp.where(ckidx == cp, _NEG, cm)
    cpos_ref[...] = cpos8

    sub8 = lax.broadcasted_iota(jnp.int32, (8, 128), 0)

    def gath(g, carry):
        rbase = pl.multiple_of(g * 8, 8)
        for j in range(K):
            acc = jnp.zeros((8, 128), jnp.float32)
            for r in range(8):
                c = cpos_ref[g * 8 + r, j]
                coff = pl.multiple_of(c * 128, 128)
                blk = sbuf_ref[pl.ds(rbase, 8), pl.ds(coff, 128)]
                acc = jnp.where(sub8 == r, blk, acc)
            cand_ref[pl.ds(rbase, 8), pl.ds(j * 128, 128)] = acc
        return carry
    lax.fori_loop(0, BR // 8, gath, 0)

    vals8 = jnp.zeros((BR, K), jnp.float32)
    idx8 = jnp.zeros((BR, K), jnp.int32)
    gidx = (jnp.broadcast_to(cpos8[:, :, None], (BR, K, 128)) * 128
            + lax.broadcasted_iota(jnp.int32, (BR, K, 128), 2)
            ).reshape(BR, K * 128)
    cand = cand_ref[...]
    for t in range(K):
        m = jnp.max(cand, axis=1, keepdims=True)
        pos = jnp.min(jnp.where(cand == m, gidx, 2 ** 30), axis=1,
                      keepdims=True)
        vals8 = jnp.where(lane8 == t, m, vals8)
        idx8 = jnp.where(lane8 == t, pos, idx8)
        if t < K - 1:
            cand = jnp.where(gidx == pos, _NEG, cand)
    rowv = lax.broadcasted_iota(jnp.int32, (BR, K), 0) + i * BR
    valid = rowv < N_REAL
    vals_ref[...] = jnp.where(valid, vals8, 0.0)
    idx_ref[...] = jnp.where(valid, idx8, 0)


def _run_simtopk(h_sim):
    return pl.pallas_call(
        _simtopk_body,
        grid=(NBLK,),
        in_specs=[pl.BlockSpec((BR, H), lambda i: (i, 0)),
                  pl.BlockSpec((NP, H), lambda i: (0, 0))],
        out_specs=[pl.BlockSpec((BR, K), lambda i: (i, 0)),
                   pl.BlockSpec((BR, K), lambda i: (i, 0))],
        out_shape=[jax.ShapeDtypeStruct((NP, K), jnp.float32),
                   jax.ShapeDtypeStruct((NP, K), jnp.int32)],
        scratch_shapes=[pltpu.VMEM((BR, NP), jnp.float32),
                        pltpu.VMEM((BR, K), jnp.int32),
                        pltpu.VMEM((BR, K * 128), jnp.float32)],
    )(h_sim, h_sim)


def _deg_body(idx_hbm, vals_hbm, out_hbm, idx_v, vals_v, deg_v):
    cid = lax.axis_index("c")
    sid = lax.axis_index("s")
    wid = cid * TPS + sid
    base = wid * EPW
    pltpu.sync_copy(idx_hbm.at[pl.ds(base, EPW)], idx_v)
    pltpu.sync_copy(vals_hbm.at[pl.ds(base, EPW)], vals_v)
    zero = jnp.zeros((16,), jnp.float32)

    def zb(j, carry):
        deg_v[pl.ds(j * 16, 16)] = zero
        return carry
    lax.fori_loop(0, 2 * NP // 16, zb, 0)

    lane = lax.iota(jnp.int32, 16)
    off = jnp.where(lane >= 8, NP, 0).astype(jnp.int32)

    def body(g, carry):
        d16 = idx_v[pl.ds(g * 16, 16)]
        w16 = vals_v[pl.ds(g * 16, 16)]
        plsc.addupdate_scatter(deg_v, [d16 + off], w16)
        return carry
    lax.fori_loop(0, GPW, body, 0)

    def fold(j, carry):
        a = deg_v[pl.ds(j * 16, 16)]
        b = deg_v[pl.ds(NP + j * 16, 16)]
        deg_v[pl.ds(j * 16, 16)] = a + b
        return carry
    lax.fori_loop(0, NP // 16, fold, 0)
    pltpu.sync_copy(deg_v.at[pl.ds(0, NP)], out_hbm.at[pl.ds(wid * NP, NP)])


def _run_deg(idx_flat, vals_flat):
    mesh = plsc.VectorSubcoreMesh(core_axis_name="c", subcore_axis_name="s")
    kfn = pl.kernel(
        _deg_body,
        out_type=jax.ShapeDtypeStruct((NW * NP,), jnp.float32),
        mesh=mesh,
        scratch_types=[pltpu.VMEM((EPW,), jnp.int32),
                       pltpu.VMEM((EPW,), jnp.float32),
                       pltpu.VMEM((2 * NP,), jnp.float32)],
        compiler_params=pltpu.CompilerParams(needs_layout_passes=False),
    )
    return kfn(idx_flat, vals_flat).reshape(NW, NP)


def _dinv_body(degp_ref, dinv_ref):
    j = pl.program_id(0)
    dsum = jnp.sum(degp_ref[...], axis=0, keepdims=True)
    col = lax.broadcasted_iota(jnp.int32, (1, 1024), 1) + j * 1024
    deg = dsum + jnp.where(col < N_REAL, 1.0, 0.0)
    dinv_ref[...] = jnp.where(deg > 0, lax.rsqrt(deg), 0.0)


def _run_dinv(deg_p):
    return pl.pallas_call(
        _dinv_body,
        grid=(NP // 1024,),
        in_specs=[pl.BlockSpec((NW, 1024), lambda j: (0, j))],
        out_specs=pl.BlockSpec((1, 1024), lambda j: (0, j)),
        out_shape=jax.ShapeDtypeStruct((1, NP), jnp.float32),
    )(deg_p)


FPT = H // NW
ECH = 8192
NWIN = NP * K // ECH


def _msg_body(xwt_hbm, idx_hbm, vals_hbm, dinv_hbm, out_hbm,
              xt_v, idx_v, vals_v, dinv_v, acc_v):
    cid = lax.axis_index("c")
    sid = lax.axis_index("s")
    wid = cid * TPS + sid
    fbase = wid * FPT
    pltpu.sync_copy(dinv_hbm, dinv_v)
    for f in range(FPT):
        pltpu.sync_copy(xwt_hbm.at[pl.ds((fbase + f) * NP, NP)],
                        xt_v.at[pl.ds(f * NP, NP)])

    zero = jnp.zeros((16,), jnp.float32)

    def zb(j, carry):
        acc_v[pl.ds(j * 16, 16)] = zero
        return carry
    lax.fori_loop(0, FPT * 2 * NP // 16, zb, 0)

    lane = lax.iota(jnp.int32, 16)
    off = jnp.where(lane >= 8, NP, 0).astype(jnp.int32)

    def win(w, carry):
        pltpu.sync_copy(idx_hbm.at[pl.ds(w * ECH, ECH)], idx_v)
        pltpu.sync_copy(vals_hbm.at[pl.ds(w * ECH, ECH)], vals_v)

        def body(g, carry2):
            e0 = g * 16
            d16 = idx_v[pl.ds(e0, 16)]
            w16 = vals_v[pl.ds(e0, 16)]
            src = lax.shift_right_logical(w * ECH + e0 + lane, 3)
            dsrc = plsc.load_gather(dinv_v, [src])
            ddst = plsc.load_gather(dinv_v, [d16])
            scale = dsrc * w16 * ddst
            tgt = d16 + off
            for f in range(FPT):
                xg = plsc.load_gather(xt_v, [src + f * NP])
                plsc.addupdate_scatter(acc_v, [tgt + f * 2 * NP],
                                       scale * xg)
            return carry2
        lax.fori_loop(0, ECH // 16, body, 0, unroll=4)
        return carry
    lax.fori_loop(0, NWIN, win, 0)

    def fold(j, carry):
        for f in range(FPT):
            a = acc_v[pl.ds(f * 2 * NP + j * 16, 16)]
            b = acc_v[pl.ds(f * 2 * NP + NP + j * 16, 16)]
            acc_v[pl.ds(f * 2 * NP + j * 16, 16)] = a + b
        return carry
    lax.fori_loop(0, NP // 16, fold, 0)
    for f in range(FPT):
        pltpu.sync_copy(acc_v.at[pl.ds(f * 2 * NP, NP)],
                        out_hbm.at[pl.ds((fbase + f) * NP, NP)])


def _run_msg(xwt_flat, idx_flat, vals_flat, dinv):
    mesh = plsc.VectorSubcoreMesh(core_axis_name="c", subcore_axis_name="s")
    kfn = pl.kernel(
        _msg_body,
        out_type=jax.ShapeDtypeStruct((H * NP,), jnp.float32),
        mesh=mesh,
        scratch_types=[pltpu.VMEM((FPT * NP,), jnp.float32),
                       pltpu.VMEM((ECH,), jnp.int32),
                       pltpu.VMEM((ECH,), jnp.float32),
                       pltpu.VMEM((NP,), jnp.float32),
                       pltpu.VMEM((FPT * 2 * NP,), jnp.float32)],
        compiler_params=pltpu.CompilerParams(needs_layout_passes=False),
    )
    return kfn(xwt_flat, idx_flat, vals_flat, dinv).reshape(H, NP)


def _combine_body(p_ref, xw_ref, dinv_ref, b_ref, w_ref, out_ref, outt_ref):
    p = jnp.transpose(p_ref[...], (1, 0))
    dv = dinv_ref[...]
    o = p + dv * dv * xw_ref[...] + b_ref[...]
    o = jnp.maximum(o, 0.0)
    xw2 = lax.dot_general(o, w_ref[...], (((1,), (1,)), ((), ())))
    out_ref[...] = xw2
    outt_ref[...] = jnp.transpose(xw2, (1, 0))


def _run_combine(agg_t, xw, dinv_col, b, w):
    return pl.pallas_call(
        _combine_body,
        grid=(NBLK,),
        in_specs=[pl.BlockSpec((H, BR), lambda i: (0, i)),
                  pl.BlockSpec((BR, H), lambda i: (i, 0)),
                  pl.BlockSpec((BR, 1), lambda i: (i, 0)),
                  pl.BlockSpec((1, H), lambda i: (0, 0)),
                  pl.BlockSpec(w.shape, lambda i: (0, 0))],
        out_specs=[pl.BlockSpec((BR, H), lambda i: (i, 0)),
                   pl.BlockSpec((H, BR), lambda i: (0, i))],
        out_shape=[jax.ShapeDtypeStruct((NP, H), jnp.float32),
                   jax.ShapeDtypeStruct((H, NP), jnp.float32)],
    )(agg_t, xw, dinv_col, b, w)


def _final_relu_body(p_ref, xw_ref, dinv_ref, b_ref, out_ref, st_ref):
    i = pl.program_id(0)
    p = jnp.transpose(p_ref[...], (1, 0))
    dv = dinv_ref[...]
    o = p + dv * dv * xw_ref[...] + b_ref[...]
    o = jnp.maximum(o, 0.0)
    out_ref[...] = o
    rowv = lax.broadcasted_iota(jnp.int32, (BR, H), 0) + i * BR
    om = jnp.where(rowv < N_REAL, o, 0.0)
    st_ref[0, 0, :] = jnp.sum(om, axis=0)
    st_ref[0, 1, :] = jnp.sum(om * om, axis=0)


def _run_final_relu(agg_t, xw, dinv_col, b):
    return pl.pallas_call(
        _final_relu_body,
        grid=(NBLK,),
        in_specs=[pl.BlockSpec((H, BR), lambda i: (0, i)),
                  pl.BlockSpec((BR, H), lambda i: (i, 0)),
                  pl.BlockSpec((BR, 1), lambda i: (i, 0)),
                  pl.BlockSpec((1, H), lambda i: (0, 0))],
        out_specs=[pl.BlockSpec((BR, H), lambda i: (i, 0)),
                   pl.BlockSpec((1, 2, H), lambda i: (i, 0, 0))],
        out_shape=[jax.ShapeDtypeStruct((NP, H), jnp.float32),
                   jax.ShapeDtypeStruct((NBLK, 2, H), jnp.float32)],
    )(agg_t, xw, dinv_col, b)


def _norm_proj_body(o2_ref, st_ref, gamma_ref, beta_ref, wp_ref, bp_ref,
                    out_ref):
    st = jnp.sum(st_ref[...], axis=0)
    cnt = jnp.float32(N_REAL)
    mean = st[0:1, :] / cnt
    var = st[1:2, :] / cnt - mean * mean
    o2 = o2_ref[...]
    o2n = (o2 - mean) / jnp.sqrt(var + 1e-5) * gamma_ref[...] + beta_ref[...]
    out_ref[...] = (lax.dot_general(o2n, wp_ref[...], (((1,), (1,)), ((), ())))
                    + bp_ref[...])


def _run_norm_proj(out2, stats, gamma, beta, wp, bp):
    return pl.pallas_call(
        _norm_proj_body,
        grid=(NBLK,),
        in_specs=[pl.BlockSpec((BR, H), lambda i: (i, 0)),
                  pl.BlockSpec((NBLK, 2, H), lambda i: (0, 0, 0)),
                  pl.BlockSpec((1, H), lambda i: (0, 0)),
                  pl.BlockSpec((1, H), lambda i: (0, 0)),
                  pl.BlockSpec((O_DIM, H), lambda i: (0, 0)),
                  pl.BlockSpec((1, O_DIM), lambda i: (0, 0))],
        out_specs=pl.BlockSpec((BR, O_DIM), lambda i: (i, 0)),
        out_shape=jax.ShapeDtypeStruct((NP, O_DIM), jnp.float32),
    )(out2, stats, gamma, beta, wp, bp)


def _split_gru_weights(wih, whh, bih, bhh):
    wir, wiz, win = jnp.split(wih, 3, axis=0)
    whr, whz, whn = jnp.split(whh, 3, axis=0)
    bir, biz, bin_ = jnp.split(bih, 3)
    bhr, bhz, bhn = jnp.split(bhh, 3)
    r2 = lambda a: a.reshape(1, H)
    return [wir, whr, r2(bir), r2(bhr), wiz, whz, r2(biz), r2(bhz),
            win, whn, r2(bin_), r2(bhn)]


@jax.jit
def kernel(x_seq, Wih_sim, Whh_sim, bih_sim, bhh_sim, Wih, Whh, bih, bhh,
           W1, b1, W2, b2, gamma, beta, Wp, bp):
    x0 = x_seq.reshape(4, F_IN, N_REAL)
    xp = jnp.pad(x0, ((0, 0), (0, 0), (0, NP - N_REAL)))

    wsets = (_split_gru_weights(Wih_sim, Whh_sim, bih_sim, bhh_sim)
             + _split_gru_weights(Wih, Whh, bih, bhh))
    h_sim, xw1, xw1_t = _run_gru(xp, wsets, W1)

    vals, idx = _run_simtopk(h_sim)
    idx_flat = idx.reshape(NP * K)
    vals_flat = vals.reshape(NP * K)

    deg_p = _run_deg(idx_flat, vals_flat)
    dinv = _run_dinv(deg_p)
    dinv_flat = dinv.reshape(NP)
    dinv_col = dinv.reshape(NP, 1)

    p1 = _run_msg(xw1_t.reshape(H * NP), idx_flat, vals_flat, dinv_flat)
    xw2, xw2_t = _run_combine(p1, xw1, dinv_col, b1.reshape(1, H), W2)

    p2 = _run_msg(xw2_t.reshape(H * NP), idx_flat, vals_flat, dinv_flat)
    out2, stats = _run_final_relu(p2, xw2, dinv_col, b2.reshape(1, H))

    out = _run_norm_proj(out2, stats, gamma.reshape(1, H),
                         beta.reshape(1, H), Wp, bp.reshape(1, O_DIM))
    return out[:N_REAL].reshape(1, N_REAL, O_DIM)


# speedup vs baseline: 35.3660x; 1.0003x over previous
import jax
import jax.numpy as jnp
from jax import lax
from jax.experimental import pallas as pl
from jax.experimental.pallas import tpu as pltpu
from jax.experimental.pallas import tpu_sc as plsc

N_REAL = 10000
NP = 10240
K = 8
F_IN = 128
H = 64
O_DIM = 32
BR = 256
NBLK = NP // BR
NW = 32
RPW = NP // NW
EPW = RPW * K
GPW = EPW // 16
TPS = 16
RPT = NP // TPS

_NEG = -3e38


def _gru_body(x_ref, *refs):
    (wir_s, whr_s, br_s, bhr_s, wiz_s, whz_s, bz_s, bhz_s, win_s, whn_s,
     bn_s, bhn_s,
     wir_o, whr_o, br_o, bhr_o, wiz_o, whz_o, bz_o, bhz_o, win_o, whn_o,
     bn_o, bhn_o,
     w1, hs_ref, xw1_ref, xw1t_ref) = refs

    def one_gru(wir, whr, bir, bhr, wiz, whz, biz, bhz, win, whn, bin_, bhn):
        h = jnp.zeros((BR, H), jnp.float32)
        for l in range(4):
            xl = x_ref[l]
            dn = (((0,), (1,)), ((), ()))
            gir = lax.dot_general(xl, wir[...], dn) + bir[...]
            giz = lax.dot_general(xl, wiz[...], dn) + biz[...]
            gin = lax.dot_general(xl, win[...], dn) + bin_[...]
            dnh = (((1,), (1,)), ((), ()))
            ghr = lax.dot_general(h, whr[...], dnh) + bhr[...]
            ghz = lax.dot_general(h, whz[...], dnh) + bhz[...]
            ghn = lax.dot_general(h, whn[...], dnh) + bhn[...]
            r = jax.nn.sigmoid(gir + ghr)
            z = jax.nn.sigmoid(giz + ghz)
            n = jnp.tanh(gin + r * ghn)
            h = (1.0 - z) * n + z * h
        return h

    h_sim = one_gru(wir_s, whr_s, br_s, bhr_s, wiz_s, whz_s, bz_s, bhz_s,
                    win_s, whn_s, bn_s, bhn_s)
    h_out = one_gru(wir_o, whr_o, br_o, bhr_o, wiz_o, whz_o, bz_o, bhz_o,
                    win_o, whn_o, bn_o, bhn_o)
    hs_ref[...] = h_sim
    xw1 = lax.dot_general(h_out, w1[...], (((1,), (1,)), ((), ())))
    xw1_ref[...] = xw1
    xw1t_ref[...] = jnp.transpose(xw1, (1, 0))


def _run_gru(xp, wsets, w1):
    full = lambda a: pl.BlockSpec(a.shape, lambda i: (0,) * a.ndim)
    in_specs = [pl.BlockSpec((4, F_IN, BR), lambda i: (0, 0, i))]
    in_specs += [full(a) for a in wsets] + [full(w1)]
    out_specs = [pl.BlockSpec((BR, H), lambda i: (i, 0)),
                 pl.BlockSpec((BR, H), lambda i: (i, 0)),
                 pl.BlockSpec((H, BR), lambda i: (0, i))]
    return pl.pallas_call(
        _gru_body,
        grid=(NBLK,),
        in_specs=in_specs,
        out_specs=out_specs,
        out_shape=[jax.ShapeDtypeStruct((NP, H), jnp.float32),
                   jax.ShapeDtypeStruct((NP, H), jnp.float32),
                   jax.ShapeDtypeStruct((H, NP), jnp.float32)],
    )(xp, *wsets, w1)


NCHUNK = NP // 128


def _simtopk_body(hrow_ref, hall_ref, vals_ref, idx_ref,
                  sbuf_ref, cpos_ref, cand_ref):
    i = pl.program_id(0)
    hrow = hrow_ref[...]
    bc = 2048
    cm_parts = []
    for c in range(NP // bc):
        hc = hall_ref[pl.ds(c * bc, bc), :]
        s = lax.dot_general(hrow, hc, (((1,), (1,)), ((), ())))
        col = lax.broadcasted_iota(jnp.int32, (BR, bc), 1) + c * bc
        row = lax.broadcasted_iota(jnp.int32, (BR, bc), 0) + i * BR
        s = jnp.where(col == row, -1e9, s)
        s = jnp.where(col >= N_REAL, _NEG, s)
        sbuf_ref[:, pl.ds(c * bc, bc)] = s
        cm_parts.append(jnp.max(s.reshape(BR, bc // 128, 128), axis=2))

    cm = jnp.concatenate(cm_parts, axis=1)
    ckidx = lax.broadcasted_iota(jnp.int32, (BR, NCHUNK), 1)
    lane8 = lax.broadcasted_iota(jnp.int32, (BR, K), 1)
    cpos8 = jnp.zeros((BR, K), jnp.int32)
    for t in range(K):
        m = jnp.max(cm, axis=1, keepdims=True)
        cp = jnp.min(jnp.where(cm == m, ckidx, 2 ** 30), axis=1,
                     keepdims=True)
        cpos8 = jnp.where(lane8 == t, cp, cpos8)
        if t < K - 1:
            cm = jnp.where(ckidx == cp, _NEG, cm)
    cpos_ref[...] = cpos8

    sub8 = lax.broadcasted_iota(jnp.int32, (8, 128), 0)

    def gath(g, carry):
        rbase = pl.multiple_of(g * 8, 8)
        for j in range(K):
            acc = jnp.zeros((8, 128), jnp.float32)
            for r in range(8):
                c = cpos_ref[g * 8 + r, j]
                coff = pl.multiple_of(c * 128, 128)
                blk = sbuf_ref[pl.ds(rbase, 8), pl.ds(coff, 128)]
                acc = jnp.where(sub8 == r, blk, acc)
            cand_ref[pl.ds(rbase, 8), pl.ds(j * 128, 128)] = acc
        return carry
    lax.fori_loop(0, BR // 8, gath, 0)

    vals8 = jnp.zeros((BR, K), jnp.float32)
    idx8 = jnp.zeros((BR, K), jnp.int32)
    gidx = (jnp.broadcast_to(cpos8[:, :, None], (BR, K, 128)) * 128
            + lax.broadcasted_iota(jnp.int32, (BR, K, 128), 2)
            ).reshape(BR, K * 128)
    cand = cand_ref[...]
    for t in range(K):
        m = jnp.max(cand, axis=1, keepdims=True)
        pos = jnp.min(jnp.where(cand == m, gidx, 2 ** 30), axis=1,
                      keepdims=True)
        vals8 = jnp.where(lane8 == t, m, vals8)
        idx8 = jnp.where(lane8 == t, pos, idx8)
        if t < K - 1:
            cand = jnp.where(gidx == pos, _NEG, cand)
    rowv = lax.broadcasted_iota(jnp.int32, (BR, K), 0) + i * BR
    valid = rowv < N_REAL
    vals_ref[...] = jnp.where(valid, vals8, 0.0)
    idx_ref[...] = jnp.where(valid, idx8, 0)


def _run_simtopk(h_sim):
    return pl.pallas_call(
        _simtopk_body,
        grid=(NBLK,),
        in_specs=[pl.BlockSpec((BR, H), lambda i: (i, 0)),
                  pl.BlockSpec((NP, H), lambda i: (0, 0))],
        out_specs=[pl.BlockSpec((BR, K), lambda i: (i, 0)),
                   pl.BlockSpec((BR, K), lambda i: (i, 0))],
        out_shape=[jax.ShapeDtypeStruct((NP, K), jnp.float32),
                   jax.ShapeDtypeStruct((NP, K), jnp.int32)],
        scratch_shapes=[pltpu.VMEM((BR, NP), jnp.float32),
                        pltpu.VMEM((BR, K), jnp.int32),
                        pltpu.VMEM((BR, K * 128), jnp.float32)],
    )(h_sim, h_sim)


def _deg_body(idx_hbm, vals_hbm, out_hbm, idx_v, vals_v, deg_v):
    cid = lax.axis_index("c")
    sid = lax.axis_index("s")
    wid = cid * TPS + sid
    base = wid * EPW
    pltpu.sync_copy(idx_hbm.at[pl.ds(base, EPW)], idx_v)
    pltpu.sync_copy(vals_hbm.at[pl.ds(base, EPW)], vals_v)
    zero = jnp.zeros((16,), jnp.float32)

    def zb(j, carry):
        deg_v[pl.ds(j * 16, 16)] = zero
        return carry
    lax.fori_loop(0, 2 * NP // 16, zb, 0)

    lane = lax.iota(jnp.int32, 16)
    off = jnp.where(lane >= 8, NP, 0).astype(jnp.int32)

    def body(g, carry):
        d16 = idx_v[pl.ds(g * 16, 16)]
        w16 = vals_v[pl.ds(g * 16, 16)]
        plsc.addupdate_scatter(deg_v, [d16 + off], w16)
        return carry
    lax.fori_loop(0, GPW, body, 0)

    def fold(j, carry):
        a = deg_v[pl.ds(j * 16, 16)]
        b = deg_v[pl.ds(NP + j * 16, 16)]
        deg_v[pl.ds(j * 16, 16)] = a + b
        return carry
    lax.fori_loop(0, NP // 16, fold, 0)
    pltpu.sync_copy(deg_v.at[pl.ds(0, NP)], out_hbm.at[pl.ds(wid * NP, NP)])


def _run_deg(idx_flat, vals_flat):
    mesh = plsc.VectorSubcoreMesh(core_axis_name="c", subcore_axis_name="s")
    kfn = pl.kernel(
        _deg_body,
        out_type=jax.ShapeDtypeStruct((NW * NP,), jnp.float32),
        mesh=mesh,
        scratch_types=[pltpu.VMEM((EPW,), jnp.int32),
                       pltpu.VMEM((EPW,), jnp.float32),
                       pltpu.VMEM((2 * NP,), jnp.float32)],
        compiler_params=pltpu.CompilerParams(needs_layout_passes=False),
    )
    return kfn(idx_flat, vals_flat).reshape(NW, NP)


def _dinv_body(degp_ref, dinv_ref):
    j = pl.program_id(0)
    dsum = jnp.sum(degp_ref[...], axis=0, keepdims=True)
    col = lax.broadcasted_iota(jnp.int32, (1, 1024), 1) + j * 1024
    deg = dsum + jnp.where(col < N_REAL, 1.0, 0.0)
    dinv_ref[...] = jnp.where(deg > 0, lax.rsqrt(deg), 0.0)


def _run_dinv(deg_p):
    return pl.pallas_call(
        _dinv_body,
        grid=(NP // 1024,),
        in_specs=[pl.BlockSpec((NW, 1024), lambda j: (0, j))],
        out_specs=pl.BlockSpec((1, 1024), lambda j: (0, j)),
        out_shape=jax.ShapeDtypeStruct((1, NP), jnp.float32),
    )(deg_p)


FPT = H // NW
ECH = 8192
NWIN = NP * K // ECH


def _msg_body(xwt_hbm, idx_hbm, vals_hbm, dinv_hbm, out_hbm,
              xt_v, idx_v, vals_v, dinv_v, acc_v):
    cid = lax.axis_index("c")
    sid = lax.axis_index("s")
    wid = cid * TPS + sid
    fbase = wid * FPT
    pltpu.sync_copy(dinv_hbm, dinv_v)
    for f in range(FPT):
        pltpu.sync_copy(xwt_hbm.at[pl.ds((fbase + f) * NP, NP)],
                        xt_v.at[pl.ds(f * NP, NP)])

    zero = jnp.zeros((16,), jnp.float32)

    def zb(j, carry):
        acc_v[pl.ds(j * 16, 16)] = zero
        return carry
    lax.fori_loop(0, FPT * 2 * NP // 16, zb, 0)

    lane = lax.iota(jnp.int32, 16)
    off = jnp.where(lane >= 8, NP, 0).astype(jnp.int32)

    def win(w, carry):
        pltpu.sync_copy(idx_hbm.at[pl.ds(w * ECH, ECH)], idx_v)
        pltpu.sync_copy(vals_hbm.at[pl.ds(w * ECH, ECH)], vals_v)

        def body(g, carry2):
            e0 = g * 16
            d16 = idx_v[pl.ds(e0, 16)]
            w16 = vals_v[pl.ds(e0, 16)]
            src = lax.shift_right_logical(w * ECH + e0 + lane, 3)
            dsrc = plsc.load_gather(dinv_v, [src])
            ddst = plsc.load_gather(dinv_v, [d16])
            scale = dsrc * w16 * ddst
            tgt = d16 + off
            for f in range(FPT):
                xg = plsc.load_gather(xt_v, [src + f * NP])
                plsc.addupdate_scatter(acc_v, [tgt + f * 2 * NP],
                                       scale * xg)
            return carry2
        lax.fori_loop(0, ECH // 16, body, 0, unroll=4)
        return carry
    lax.fori_loop(0, NWIN, win, 0)

    def fold(j, carry):
        for f in range(FPT):
            a = acc_v[pl.ds(f * 2 * NP + j * 16, 16)]
            b = acc_v[pl.ds(f * 2 * NP + NP + j * 16, 16)]
            acc_v[pl.ds(f * 2 * NP + j * 16, 16)] = a + b
        return carry
    lax.fori_loop(0, NP // 16, fold, 0)
    for f in range(FPT):
        pltpu.sync_copy(acc_v.at[pl.ds(f * 2 * NP, NP)],
                        out_hbm.at[pl.ds((fbase + f) * NP, NP)])


def _run_msg(xwt_flat, idx_flat, vals_flat, dinv):
    mesh = plsc.VectorSubcoreMesh(core_axis_name="c", subcore_axis_name="s")
    kfn = pl.kernel(
        _msg_body,
        out_type=jax.ShapeDtypeStruct((H * NP,), jnp.float32),
        mesh=mesh,
        scratch_types=[pltpu.VMEM((FPT * NP,), jnp.float32),
                       pltpu.VMEM((ECH,), jnp.int32),
                       pltpu.VMEM((ECH,), jnp.float32),
                       pltpu.VMEM((NP,), jnp.float32),
                       pltpu.VMEM((FPT * 2 * NP,), jnp.float32)],
        compiler_params=pltpu.CompilerParams(needs_layout_passes=False),
    )
    return kfn(xwt_flat, idx_flat, vals_flat, dinv).reshape(H, NP)


def _combine_body(p_ref, xw_ref, dinv_ref, b_ref, w_ref, out_ref, outt_ref):
    p = jnp.transpose(p_ref[...], (1, 0))
    dv = dinv_ref[...]
    o = p + dv * dv * xw_ref[...] + b_ref[...]
    o = jnp.maximum(o, 0.0)
    xw2 = lax.dot_general(o, w_ref[...], (((1,), (1,)), ((), ())))
    out_ref[...] = xw2
    outt_ref[...] = jnp.transpose(xw2, (1, 0))


def _run_combine(agg_t, xw, dinv_col, b, w):
    return pl.pallas_call(
        _combine_body,
        grid=(NBLK,),
        in_specs=[pl.BlockSpec((H, BR), lambda i: (0, i)),
                  pl.BlockSpec((BR, H), lambda i: (i, 0)),
                  pl.BlockSpec((BR, 1), lambda i: (i, 0)),
                  pl.BlockSpec((1, H), lambda i: (0, 0)),
                  pl.BlockSpec(w.shape, lambda i: (0, 0))],
        out_specs=[pl.BlockSpec((BR, H), lambda i: (i, 0)),
                   pl.BlockSpec((H, BR), lambda i: (0, i))],
        out_shape=[jax.ShapeDtypeStruct((NP, H), jnp.float32),
                   jax.ShapeDtypeStruct((H, NP), jnp.float32)],
    )(agg_t, xw, dinv_col, b, w)


def _final_relu_body(p_ref, xw_ref, dinv_ref, b_ref, out_ref, st_ref):
    i = pl.program_id(0)
    p = jnp.transpose(p_ref[...], (1, 0))
    dv = dinv_ref[...]
    o = p + dv * dv * xw_ref[...] + b_ref[...]
    o = jnp.maximum(o, 0.0)
    out_ref[...] = o
    rowv = lax.broadcasted_iota(jnp.int32, (BR, H), 0) + i * BR
    om = jnp.where(rowv < N_REAL, o, 0.0)
    st_ref[0, 0, :] = jnp.sum(om, axis=0)
    st_ref[0, 1, :] = jnp.sum(om * om, axis=0)


def _run_final_relu(agg_t, xw, dinv_col, b):
    return pl.pallas_call(
        _final_relu_body,
        grid=(NBLK,),
        in_specs=[pl.BlockSpec((H, BR), lambda i: (0, i)),
                  pl.BlockSpec((BR, H), lambda i: (i, 0)),
                  pl.BlockSpec((BR, 1), lambda i: (i, 0)),
                  pl.BlockSpec((1, H), lambda i: (0, 0))],
        out_specs=[pl.BlockSpec((BR, H), lambda i: (i, 0)),
                   pl.BlockSpec((1, 2, H), lambda i: (i, 0, 0))],
        out_shape=[jax.ShapeDtypeStruct((NP, H), jnp.float32),
                   jax.ShapeDtypeStruct((NBLK, 2, H), jnp.float32)],
    )(agg_t, xw, dinv_col, b)


def _norm_proj_body(o2_ref, st_ref, gamma_ref, beta_ref, wp_ref, bp_ref,
                    out_ref):
    st = jnp.sum(st_ref[...], axis=0)
    cnt = jnp.float32(N_REAL)
    mean = st[0:1, :] / cnt
    var = st[1:2, :] / cnt - mean * mean
    o2 = o2_ref[...]
    o2n = (o2 - mean) / jnp.sqrt(var + 1e-5) * gamma_ref[...] + beta_ref[...]
    out_ref[...] = (lax.dot_general(o2n, wp_ref[...], (((1,), (1,)), ((), ())))
                    + bp_ref[...])


def _run_norm_proj(out2, stats, gamma, beta, wp, bp):
    return pl.pallas_call(
        _norm_proj_body,
        grid=(NBLK,),
        in_specs=[pl.BlockSpec((BR, H), lambda i: (i, 0)),
                  pl.BlockSpec((NBLK, 2, H), lambda i: (0, 0, 0)),
                  pl.BlockSpec((1, H), lambda i: (0, 0)),
                  pl.BlockSpec((1, H), lambda i: (0, 0)),
                  pl.BlockSpec((O_DIM, H), lambda i: (0, 0)),
                  pl.BlockSpec((1, O_DIM), lambda i: (0, 0))],
        out_specs=pl.BlockSpec((BR, O_DIM), lambda i: (i, 0)),
        out_shape=jax.ShapeDtypeStruct((NP, O_DIM), jnp.float32),
    )(out2, stats, gamma, beta, wp, bp)


def _split_gru_weights(wih, whh, bih, bhh):
    wir, wiz, win = jnp.split(wih, 3, axis=0)
    whr, whz, whn = jnp.split(whh, 3, axis=0)
    bir, biz, bin_ = jnp.split(bih, 3)
    bhr, bhz, bhn = jnp.split(bhh, 3)
    r2 = lambda a: a.reshape(1, H)
    return [wir, whr, r2(bir), r2(bhr), wiz, whz, r2(biz), r2(bhz),
            win, whn, r2(bin_), r2(bhn)]


@jax.jit
def kernel(x_seq, Wih_sim, Whh_sim, bih_sim, bhh_sim, Wih, Whh, bih, bhh,
           W1, b1, W2, b2, gamma, beta, Wp, bp):
    x0 = x_seq.reshape(4, F_IN, N_REAL)
    xp = jnp.pad(x0, ((0, 0), (0, 0), (0, NP - N_REAL)))

    wsets = (_split_gru_weights(Wih_sim, Whh_sim, bih_sim, bhh_sim)
             + _split_gru_weights(Wih, Whh, bih, bhh))
    h_sim, xw1, xw1_t = _run_gru(xp, wsets, W1)

    vals, idx = _run_simtopk(h_sim)
    idx_flat = idx.reshape(NP * K)
    vals_flat = vals.reshape(NP * K)

    deg_p = _run_deg(idx_flat, vals_flat)
    dinv = _run_dinv(deg_p)
    dinv_flat = dinv.reshape(NP)
    dinv_col = dinv.reshape(NP, 1)

    p1 = _run_msg(xw1_t.reshape(H * NP), idx_flat, vals_flat, dinv_flat)
    xw2, xw2_t = _run_combine(p1, xw1, dinv_col, b1.reshape(1, H), W2)

    p2 = _run_msg(xw2_t.reshape(H * NP), idx_flat, vals_flat, dinv_flat)
    out2, stats = _run_final_relu(p2, xw2, dinv_col, b2.reshape(1, H))

    out = _run_norm_proj(out2, stats, gamma.reshape(1, H),
                         beta.reshape(1, H), Wp, bp.reshape(1, O_DIM))
    return out[:N_REAL].reshape(1, N_REAL, O_DIM)
